# baseline (speedup 1.0000x reference)
import functools

import jax
import jax.numpy as jnp
from jax import lax
from jax.experimental import pallas as pl
from jax.experimental.pallas import tpu as pltpu
from jax.experimental.pallas import tpu_sc as plsc

NBK = 14
SIDE = 7
SF = 1.7
W = 32
L = 16
UP = 32
BU = 4
NEG = -3.4e38


def _group(cls_v, off_v, prop_v, bbox_b, conf_v, b, gu, pp,
           ub, cfb, ppb, cs_stride, prop_flat=False):
    iota = lax.iota(jnp.int32, L)
    bvec = jnp.full((L,), b, jnp.int32)
    up = gu * UP + pp
    col = up * 28 + iota * 28
    c = [plsc.load_gather(cls_v, [bvec, col + j]) for j in range(28)]

    ifs = []
    osel_s = []
    conf = None
    for s in range(4):
        e = [jnp.exp(c[SIDE * s + j]) for j in range(SIDE)]
        m1 = e[0]
        i1 = jnp.zeros((L,), jnp.float32)
        m2 = jnp.zeros((L,), jnp.float32)
        i2 = jnp.zeros((L,), jnp.float32)
        den = e[0]
        for j in range(1, SIDE):
            jv = jnp.float32(j)
            gt1 = e[j] > m1
            gt2 = e[j] > m2
            m2 = jnp.where(gt1, m1, jnp.where(gt2, e[j], m2))
            i2 = jnp.where(gt1, i1, jnp.where(gt2, jv, i2))
            m1 = jnp.maximum(e[j], m1)
            i1 = jnp.where(gt1, jv, i1)
            den = den + e[j]
        i1i = i1.astype(jnp.int32)
        osel = plsc.load_gather(off_v, [bvec, col + jnp.int32(SIDE * s) + i1i])
        rden = jnp.float32(1.0) / den
        v1 = m1 * rden
        v2 = m2 * rden
        cf = v1 + v2 * (jnp.abs(i1 - i2) - jnp.float32(1.0))
        conf = cf if conf is None else conf + cf
        ifs.append(i1)
        osel_s.append(osel)

    if prop_flat:
        fcol = b * (UP * 4) + (pp + iota) * 4
        p0 = plsc.load_gather(prop_v, [fcol])
        p1 = plsc.load_gather(prop_v, [fcol + 1])
        p2 = plsc.load_gather(prop_v, [fcol + 2])
        p3 = plsc.load_gather(prop_v, [fcol + 3])
    else:
        cvec = [jnp.full((L,), cc, jnp.int32) for cc in range(4)]
        pcol = ppb + up + iota
        p0 = plsc.load_gather(prop_v, [bvec, cvec[0], pcol])
        p1 = plsc.load_gather(prop_v, [bvec, cvec[1], pcol])
        p2 = plsc.load_gather(prop_v, [bvec, cvec[2], pcol])
        p3 = plsc.load_gather(prop_v, [bvec, cvec[3], pcol])
    cx2 = p0 + p2
    cy2 = p1 + p3
    w17 = (p2 - p0) * SF
    h17 = (p3 - p1) * SF
    px1 = (cx2 - w17) * 0.5
    px2 = (cx2 + w17) * 0.5
    py1 = (cy2 - h17) * 0.5
    py2 = (cy2 + h17) * 0.5
    bw = w17 / NBK
    bh = h17 / NBK
    half = jnp.float32(0.5)
    x1 = px1 + (half + ifs[0] - osel_s[0]) * bw
    x2 = px2 - (half + ifs[1] + osel_s[1]) * bw
    y1 = py1 + (half + ifs[2] - osel_s[2]) * bh
    y2 = py2 - (half + ifs[3] + osel_s[3]) * bh
    kvec = [jnp.full((L,), cc, jnp.int32) for cc in range(4)]
    ucol = ub + up + iota
    plsc.store_scatter(bbox_b, [bvec, kvec[0], ucol], x1)
    plsc.store_scatter(bbox_b, [bvec, kvec[1], ucol], y1)
    plsc.store_scatter(bbox_b, [bvec, kvec[2], ucol], x2)
    plsc.store_scatter(bbox_b, [bvec, kvec[3], ucol], y2)
    conf_v[pl.ds(b * cs_stride + cfb + up, L)] = conf * jnp.float32(0.25)


@functools.lru_cache(maxsize=None)
def _build(B, N):
    CU = UP * 28
    CB = CU * BU
    CH = CB // 2
    HU = BU // 2
    PB = UP * BU
    NB = N // PB
    TAIL = N - NB * PB
    K = NB // W
    R = NB - K * W
    MX = K + 1
    PW = MX * PB
    CS = PW + UP

    mesh = plsc.VectorSubcoreMesh(core_axis_name="c", subcore_axis_name="s")

    @functools.partial(
        pl.kernel,
        out_type=[
            jax.ShapeDtypeStruct((B, 4, NB * PB + PB), jnp.float32),
            jax.ShapeDtypeStruct((B * N,), jnp.float32),
        ],
        mesh=mesh,
        compiler_params=pltpu.CompilerParams(needs_layout_passes=False),
        scratch_types=[
            pltpu.VMEM((B, CH), jnp.float32),
            pltpu.VMEM((B, CH), jnp.float32),
            pltpu.VMEM((B, CH), jnp.float32),
            pltpu.VMEM((B, CH), jnp.float32),
            pltpu.VMEM((B, CH), jnp.float32),
            pltpu.VMEM((B, CH), jnp.float32),
            pltpu.VMEM((B, 4, PW), jnp.float32),
            pltpu.VMEM((B, 4, PB), jnp.float32),
            pltpu.VMEM((B * CS,), jnp.float32),
            pltpu.VMEM((B * UP * 4,), jnp.float32),
            pltpu.SemaphoreType.DMA,
            pltpu.SemaphoreType.DMA,
            pltpu.SemaphoreType.DMA,
        ],
    )
    def run(cls_hbm, off_hbm, prop_hbm, tprop_hbm, outb_hbm, outc_hbm,
            cls0, cls1, cls2, off0, off1, off2, prop_v, bbox_b, conf_v,
            tprop_v, sem0, sem1, sem2):
        w = lax.axis_index("c") * 16 + lax.axis_index("s")
        is_big = w < R
        base = jnp.where(is_big, MX * w, K * w + R)
        cnt = jnp.where(is_big, MX, K)
        H = 2 * cnt
        has_tail = w == W - 1
        bufs = ((cls0, off0, sem0), (cls1, off1, sem1), (cls2, off2, sem2))

        def fire(h, par):
            colh = base * CB + h * CH
            cbuf, obuf, sem = bufs[par]
            pltpu.make_async_copy(
                cls_hbm.at[:, pl.ds(colh, CH)], cbuf, sem).start()
            pltpu.make_async_copy(
                off_hbm.at[:, pl.ds(colh, CH)], obuf, sem).start()

        def wait(h, par):
            colh = base * CB + h * CH
            cbuf, obuf, sem = bufs[par]
            pltpu.make_async_copy(
                cls_hbm.at[:, pl.ds(colh, CH)], cbuf, sem).wait()
            pltpu.make_async_copy(
                off_hbm.at[:, pl.ds(colh, CH)], obuf, sem).wait()

        fire(jnp.int32(0), 0)
        fire(jnp.int32(1), 1)
        fire(jnp.int32(2), 2)

        @pl.when(is_big)
        def _():
            pltpu.sync_copy(prop_hbm.at[:, :, pl.ds(base * PB, MX * PB)],
                            prop_v)

        @pl.when(jnp.logical_not(is_big))
        def _():
            pltpu.sync_copy(prop_hbm.at[:, :, pl.ds(base * PB, K * PB)],
                            prop_v.at[:, :, pl.ds(0, K * PB)])

        @pl.when(has_tail)
        def _():
            pltpu.sync_copy(tprop_hbm, tprop_v)

        def process(h, par, sub):
            lb = h // 2
            blk = base + lb
            wait(h, par)
            cbuf, obuf, _ = bufs[par]

            def group_body(g, _):
                _group(cbuf, obuf, prop_v, bbox_b, conf_v,
                       (g // 2) % B, g // (2 * B), (g % 2) * L,
                       sub * (HU * UP), lb * PB + sub * (HU * UP),
                       lb * PB + sub * (HU * UP), CS)
                return _
            lax.fori_loop(0, 2 * B * HU, group_body, None)
            if sub == 1:
                pltpu.sync_copy(bbox_b,
                                outb_hbm.at[:, :, pl.ds(blk * PB, PB)])

            @pl.when(h + 3 < H)
            def _():
                fire(h + 3, par)

        def half_body(h, _):
            par3 = h % 3
            sub = h & 1

            for p in range(3):
                for sb in range(2):
                    @pl.when((par3 == p) & (sub == sb))
                    def _(p=p, sb=sb):
                        process(h, p, sb)
            return _
        lax.fori_loop(0, H, half_body, None)

        @pl.when(has_tail)
        def _():
            pltpu.sync_copy(cls_hbm.at[:, pl.ds(NB * CB, CU)],
                            cls0.at[:, pl.ds(0, CU)])
            pltpu.sync_copy(off_hbm.at[:, pl.ds(NB * CB, CU)],
                            off0.at[:, pl.ds(0, CU)])

            def tgroup_body(g, _):
                _group(cls0, off0, tprop_v, bbox_b, conf_v,
                       g // 2, 0, (g % 2) * L,
                       0, K * PB, 0, CS, prop_flat=True)
                return _
            lax.fori_loop(0, 2 * B, tgroup_body, None)
            pltpu.sync_copy(bbox_b, outb_hbm.at[:, :, pl.ds(NB * PB, PB)])

        @pl.when(is_big)
        def _():
            for b in range(B):
                pltpu.sync_copy(
                    conf_v.at[pl.ds(b * CS, MX * PB)],
                    outc_hbm.at[pl.ds(b * N + base * PB, MX * PB)])

        @pl.when(jnp.logical_not(is_big) & jnp.logical_not(has_tail))
        def _():
            for b in range(B):
                pltpu.sync_copy(
                    conf_v.at[pl.ds(b * CS, K * PB)],
                    outc_hbm.at[pl.ds(b * N + base * PB, K * PB)])

        @pl.when(has_tail)
        def _():
            for b in range(B):
                pltpu.sync_copy(
                    conf_v.at[pl.ds(b * CS, K * PB + TAIL)],
                    outc_hbm.at[pl.ds(b * N + base * PB, K * PB + TAIL)])

    return run


def kernel(proposals, cls_preds, offset_preds):
    B, N, _ = proposals.shape
    nfull = (N // 128) * 128
    tail_prop = proposals[:, nfull:, :].reshape(-1)
    bbox_t, conf_flat = _build(B, N)(
        cls_preds, offset_preds, jnp.swapaxes(proposals, 1, 2), tail_prop)
    return jnp.swapaxes(bbox_t, 1, 2)[:, :N, :], conf_flat.reshape(B, N)


# speedup vs baseline: 1.0493x; 1.0493x over previous
import functools

import jax
import jax.numpy as jnp
from jax import lax
from jax.experimental import pallas as pl
from jax.experimental.pallas import tpu as pltpu
from jax.experimental.pallas import tpu_sc as plsc

NBK = 14
SIDE = 7
SF = 1.7
W = 32
L = 16
UP = 32
BU = 4
NEG = -3.4e38


def _group(cls_v, off_v, prop_v, bbox_b, conf_v, b, gu, pp,
           ub, cfb, ppb, cs_stride, prop_flat=False):
    iota = lax.iota(jnp.int32, L)
    bvec = jnp.full((L,), b, jnp.int32)
    up = gu * UP + pp
    col = up * 28 + iota * 28
    c = [plsc.load_gather(cls_v, [bvec, col + j]) for j in range(28)]

    ifs = []
    osel_s = []
    conf = None
    for s in range(4):
        e = [jnp.exp(c[SIDE * s + j]) for j in range(SIDE)]
        m1 = e[0]
        i1 = jnp.zeros((L,), jnp.float32)
        m2 = jnp.zeros((L,), jnp.float32)
        i2 = jnp.zeros((L,), jnp.float32)
        den = e[0]
        for j in range(1, SIDE):
            jv = jnp.float32(j)
            gt1 = e[j] > m1
            gt2 = e[j] > m2
            m2 = jnp.where(gt1, m1, jnp.where(gt2, e[j], m2))
            i2 = jnp.where(gt1, i1, jnp.where(gt2, jv, i2))
            m1 = jnp.maximum(e[j], m1)
            i1 = jnp.where(gt1, jv, i1)
            den = den + e[j]
        i1i = i1.astype(jnp.int32)
        osel = plsc.load_gather(off_v, [bvec, col + jnp.int32(SIDE * s) + i1i])
        rden = jnp.float32(1.0) / den
        v1 = m1 * rden
        v2 = m2 * rden
        cf = v1 + v2 * (jnp.abs(i1 - i2) - jnp.float32(1.0))
        conf = cf if conf is None else conf + cf
        ifs.append(i1)
        osel_s.append(osel)

    if prop_flat:
        fcol = b * (UP * 4) + (pp + iota) * 4
        p0 = plsc.load_gather(prop_v, [fcol])
        p1 = plsc.load_gather(prop_v, [fcol + 1])
        p2 = plsc.load_gather(prop_v, [fcol + 2])
        p3 = plsc.load_gather(prop_v, [fcol + 3])
    else:
        cvec = [jnp.full((L,), cc, jnp.int32) for cc in range(4)]
        pcol = ppb + up + iota
        p0 = plsc.load_gather(prop_v, [bvec, cvec[0], pcol])
        p1 = plsc.load_gather(prop_v, [bvec, cvec[1], pcol])
        p2 = plsc.load_gather(prop_v, [bvec, cvec[2], pcol])
        p3 = plsc.load_gather(prop_v, [bvec, cvec[3], pcol])
    cx2 = p0 + p2
    cy2 = p1 + p3
    w17 = (p2 - p0) * SF
    h17 = (p3 - p1) * SF
    px1 = (cx2 - w17) * 0.5
    px2 = (cx2 + w17) * 0.5
    py1 = (cy2 - h17) * 0.5
    py2 = (cy2 + h17) * 0.5
    bw = w17 / NBK
    bh = h17 / NBK
    half = jnp.float32(0.5)
    x1 = px1 + (half + ifs[0] - osel_s[0]) * bw
    x2 = px2 - (half + ifs[1] + osel_s[1]) * bw
    y1 = py1 + (half + ifs[2] - osel_s[2]) * bh
    y2 = py2 - (half + ifs[3] + osel_s[3]) * bh
    kvec = [jnp.full((L,), cc, jnp.int32) for cc in range(4)]
    ucol = ub + up + iota
    plsc.store_scatter(bbox_b, [bvec, kvec[0], ucol], x1)
    plsc.store_scatter(bbox_b, [bvec, kvec[1], ucol], y1)
    plsc.store_scatter(bbox_b, [bvec, kvec[2], ucol], x2)
    plsc.store_scatter(bbox_b, [bvec, kvec[3], ucol], y2)
    conf_v[pl.ds(b * cs_stride + cfb + up, L)] = conf * jnp.float32(0.25)


@functools.lru_cache(maxsize=None)
def _build(B, N):
    CU = UP * 28
    CB = CU * BU
    CH = CB // 2
    HU = BU // 2
    PB = UP * BU
    NB = N // PB
    TAIL = N - NB * PB
    K = NB // W
    R = NB - K * W
    MX = K + 1
    PW = MX * PB
    CS = PW + UP

    mesh = plsc.VectorSubcoreMesh(core_axis_name="c", subcore_axis_name="s")

    @functools.partial(
        pl.kernel,
        out_type=[
            jax.ShapeDtypeStruct((B, 4, NB * PB + PB), jnp.float32),
            jax.ShapeDtypeStruct((B * N,), jnp.float32),
        ],
        mesh=mesh,
        compiler_params=pltpu.CompilerParams(needs_layout_passes=False),
        scratch_types=[
            pltpu.VMEM((B, CH), jnp.float32),
            pltpu.VMEM((B, CH), jnp.float32),
            pltpu.VMEM((B, CH), jnp.float32),
            pltpu.VMEM((B, CH), jnp.float32),
            pltpu.VMEM((B, 4, PW), jnp.float32),
            pltpu.VMEM((B, 4, PB), jnp.float32),
            pltpu.VMEM((B * CS,), jnp.float32),
            pltpu.VMEM((B * UP * 4,), jnp.float32),
            pltpu.SemaphoreType.DMA,
            pltpu.SemaphoreType.DMA,
        ],
    )
    def run(cls_hbm, off_hbm, prop_hbm, tprop_hbm, outb_hbm, outc_hbm,
            cls0, cls1, off0, off1, prop_v, bbox_b, conf_v, tprop_v,
            sem0, sem1):
        w = lax.axis_index("c") * 16 + lax.axis_index("s")
        is_big = w < R
        base = jnp.where(is_big, MX * w, K * w + R)
        cnt = jnp.where(is_big, MX, K)
        H = 2 * cnt
        has_tail = w == W - 1
        bufs = ((cls0, off0, sem0), (cls1, off1, sem1))

        def fire(h, par):
            colh = base * CB + h * CH
            cbuf, obuf, sem = bufs[par]
            pltpu.make_async_copy(
                cls_hbm.at[:, pl.ds(colh, CH)], cbuf, sem).start()
            pltpu.make_async_copy(
                off_hbm.at[:, pl.ds(colh, CH)], obuf, sem).start()

        def wait(h, par):
            colh = base * CB + h * CH
            cbuf, obuf, sem = bufs[par]
            pltpu.make_async_copy(
                cls_hbm.at[:, pl.ds(colh, CH)], cbuf, sem).wait()
            pltpu.make_async_copy(
                off_hbm.at[:, pl.ds(colh, CH)], obuf, sem).wait()

        fire(jnp.int32(0), 0)
        fire(jnp.int32(1), 1)

        @pl.when(is_big)
        def _():
            pltpu.sync_copy(prop_hbm.at[:, :, pl.ds(base * PB, MX * PB)],
                            prop_v)

        @pl.when(jnp.logical_not(is_big))
        def _():
            pltpu.sync_copy(prop_hbm.at[:, :, pl.ds(base * PB, K * PB)],
                            prop_v.at[:, :, pl.ds(0, K * PB)])

        @pl.when(has_tail)
        def _():
            pltpu.sync_copy(tprop_hbm, tprop_v)

        def process(h, par, sub):
            lb = h // 2
            blk = base + lb
            wait(h, par)
            cbuf, obuf, _ = bufs[par]

            def group_body(g, _):
                _group(cbuf, obuf, prop_v, bbox_b, conf_v,
                       (g // 2) % B, g // (2 * B), (g % 2) * L,
                       sub * (HU * UP), lb * PB + sub * (HU * UP),
                       lb * PB + sub * (HU * UP), CS)
                return _
            lax.fori_loop(0, 2 * B * HU, group_body, None)
            if sub == 1:
                pltpu.sync_copy(bbox_b,
                                outb_hbm.at[:, :, pl.ds(blk * PB, PB)])

            @pl.when(h + 2 < H)
            def _():
                fire(h + 2, par)

        def half_body(h, _):
            @pl.when((h & 1) == 0)
            def _():
                process(h, 0, 0)

            @pl.when((h & 1) == 1)
            def _():
                process(h, 1, 1)
            return _
        lax.fori_loop(0, H, half_body, None)

        @pl.when(has_tail)
        def _():
            pltpu.sync_copy(cls_hbm.at[:, pl.ds(NB * CB, CU)],
                            cls0.at[:, pl.ds(0, CU)])
            pltpu.sync_copy(off_hbm.at[:, pl.ds(NB * CB, CU)],
                            off0.at[:, pl.ds(0, CU)])

            def tgroup_body(g, _):
                _group(cls0, off0, tprop_v, bbox_b, conf_v,
                       g // 2, 0, (g % 2) * L,
                       0, K * PB, 0, CS, prop_flat=True)
                return _
            lax.fori_loop(0, 2 * B, tgroup_body, None)
            pltpu.sync_copy(bbox_b, outb_hbm.at[:, :, pl.ds(NB * PB, PB)])

        @pl.when(is_big)
        def _():
            for b in range(B):
                pltpu.sync_copy(
                    conf_v.at[pl.ds(b * CS, MX * PB)],
                    outc_hbm.at[pl.ds(b * N + base * PB, MX * PB)])

        @pl.when(jnp.logical_not(is_big) & jnp.logical_not(has_tail))
        def _():
            for b in range(B):
                pltpu.sync_copy(
                    conf_v.at[pl.ds(b * CS, K * PB)],
                    outc_hbm.at[pl.ds(b * N + base * PB, K * PB)])

        @pl.when(has_tail)
        def _():
            for b in range(B):
                pltpu.sync_copy(
                    conf_v.at[pl.ds(b * CS, K * PB + TAIL)],
                    outc_hbm.at[pl.ds(b * N + base * PB, K * PB + TAIL)])

    return run


def kernel(proposals, cls_preds, offset_preds):
    B, N, _ = proposals.shape
    nfull = (N // 128) * 128
    tail_prop = proposals[:, nfull:, :].reshape(-1)
    bbox_t, conf_flat = _build(B, N)(
        cls_preds, offset_preds, jnp.swapaxes(proposals, 1, 2), tail_prop)
    return jnp.swapaxes(bbox_t, 1, 2)[:, :N, :], conf_flat.reshape(B, N)
